# baseline (speedup 1.0000x reference)
import functools

import jax
import jax.numpy as jnp
from jax import lax
from jax.experimental import pallas as pl
from jax.experimental.pallas import tpu as pltpu
from jax.experimental.pallas import tpu_sc as plsc

N_NODES_C = 10000
IN_CH_C = 128
OUT_CH_C = 128
NNZ_C = 160000

_NC = 2
_NS = 16
_NW = _NC * _NS

_PER_W = NNZ_C // _NW
_ROWS128 = 40
_PAD_W = _ROWS128 * 128
_FLAT = N_NODES_C * IN_CH_C
_STRIPE = _FLAT // _NS
_ZB = 8000


def _sc_body(rows_hbm, cols_hbm, vals_hbm, out0_hbm, out1_hbm,
             shared, rows_v, cols_v, vals_v, flat2, vals2, zbuf,
             sem_in, sem_z, sem_sc):
    c = lax.axis_index("c")
    s = lax.axis_index("s")
    w = c * _NS + s
    base = w * _PER_W

    d_in = [
        pltpu.async_copy(rows_hbm.at[pl.ds(base, _PER_W)],
                         rows_v.at[pl.ds(0, _PER_W)], sem_in),
        pltpu.async_copy(cols_hbm.at[pl.ds(base, _PER_W)],
                         cols_v.at[pl.ds(0, _PER_W)], sem_in),
        pltpu.async_copy(vals_hbm.at[pl.ds(base, _PER_W)],
                         vals_v.at[pl.ds(0, _PER_W)], sem_in),
    ]

    zero16 = jnp.zeros((16,), jnp.float32)

    def _zb_body(i, _):
        for u in range(4):
            zbuf[pl.ds(pl.multiple_of(i * 64 + u * 16, 16), 16)] = zero16
        return _

    lax.fori_loop(0, _ZB // 64, _zb_body, None)
    d_z = [
        pltpu.async_copy(zbuf, shared.at[pl.ds(s * _STRIPE + k * _ZB, _ZB)],
                         sem_z)
        for k in range(_STRIPE // _ZB)
    ]

    for d in d_in:
        d.wait()
    iota16 = lax.iota(jnp.int32, 16)

    def _cvt_body(j, _):
        for k in range(8):
            off = pl.multiple_of(j * 128 + k * 16, 16)
            r = rows_v[pl.ds(off, 16)]
            cc = cols_v[pl.ds(off, 16)]
            flat2[j, pl.ds(k * 16, 16)] = r * IN_CH_C + cc
            vals2[j, pl.ds(k * 16, 16)] = vals_v[pl.ds(off, 16)]
        return _

    lax.fori_loop(0, _ROWS128 - 1, _cvt_body, None)
    jt = _ROWS128 - 1

    def _tail_body(k, _):
        off = pl.multiple_of(jt * 128 + k * 16, 16)
        ok = (k * 16 + iota16) < (_PER_W - jt * 128)
        r = rows_v[pl.ds(off, 16)]
        cc = cols_v[pl.ds(off, 16)]
        v = vals_v[pl.ds(off, 16)]
        flat2[jt, pl.ds(k * 16, 16)] = jnp.where(ok, r * IN_CH_C + cc, 0)
        vals2[jt, pl.ds(k * 16, 16)] = jnp.where(ok, v, jnp.float32(0.0))
        return _

    lax.fori_loop(0, 8, _tail_body, None)

    for d in d_z:
        d.wait()

    plsc.subcore_barrier()

    _PRE = 4
    d_pre = [
        pltpu.async_copy(vals2.at[j], shared.at[flat2.at[j]], sem_sc,
                         add=True)
        for j in range(_PRE)
    ]

    def _sc_scatter(j, _):
        d = pltpu.async_copy(vals2.at[j], shared.at[flat2.at[j]], sem_sc,
                             add=True)
        d.wait()
        return _

    lax.fori_loop(_PRE, _ROWS128, _sc_scatter, None)
    for d in d_pre:
        d.wait()

    plsc.subcore_barrier()

    @pl.when(c == 0)
    def _():
        pltpu.sync_copy(shared.at[pl.ds(s * _STRIPE, _STRIPE)],
                        out0_hbm.at[pl.ds(s * _STRIPE, _STRIPE)])

    @pl.when(c == 1)
    def _():
        pltpu.sync_copy(shared.at[pl.ds(s * _STRIPE, _STRIPE)],
                        out1_hbm.at[pl.ds(s * _STRIPE, _STRIPE)])


_densify = functools.partial(
    pl.kernel,
    out_type=[jax.ShapeDtypeStruct((_FLAT,), jnp.float32),
              jax.ShapeDtypeStruct((_FLAT,), jnp.float32)],
    mesh=plsc.VectorSubcoreMesh(core_axis_name="c", subcore_axis_name="s"),
    scratch_types=[
        pltpu.VMEM_SHARED((_FLAT,), jnp.float32),
        pltpu.VMEM((_PAD_W,), jnp.int32),
        pltpu.VMEM((_PAD_W,), jnp.int32),
        pltpu.VMEM((_PAD_W,), jnp.float32),
        pltpu.VMEM((_ROWS128, 128), jnp.int32),
        pltpu.VMEM((_ROWS128, 128), jnp.float32),
        pltpu.VMEM((_ZB,), jnp.float32),
        pltpu.SemaphoreType.DMA,
        pltpu.SemaphoreType.DMA,
        pltpu.SemaphoreType.DMA,
    ],
)(_sc_body)


def _mm_body(s0_ref, s1_ref, w_ref, o_ref):
    a = s0_ref[...] + s1_ref[...]
    o_ref[...] = jnp.dot(a, w_ref[...], preferred_element_type=jnp.float32)


_ROW_BLK = 1000


def _matmul(s0, s1, weight):
    return pl.pallas_call(
        _mm_body,
        grid=(N_NODES_C // _ROW_BLK,),
        in_specs=[
            pl.BlockSpec((_ROW_BLK, IN_CH_C), lambda i: (i, 0)),
            pl.BlockSpec((_ROW_BLK, IN_CH_C), lambda i: (i, 0)),
            pl.BlockSpec((IN_CH_C, OUT_CH_C), lambda i: (0, 0)),
        ],
        out_specs=pl.BlockSpec((_ROW_BLK, OUT_CH_C), lambda i: (i, 0)),
        out_shape=jax.ShapeDtypeStruct((N_NODES_C, OUT_CH_C), jnp.float32),
    )(s0, s1, weight)


def kernel(adj_indices, adj_values, feat_rows, feat_cols, feat_values, weight):
    del adj_indices, adj_values
    p0, p1 = _densify(feat_rows.astype(jnp.int32), feat_cols.astype(jnp.int32),
                      feat_values)
    s0 = p0.reshape(N_NODES_C, IN_CH_C)
    s1 = p1.reshape(N_NODES_C, IN_CH_C)
    return _matmul(s0, s1, weight)


# speedup vs baseline: 1.1133x; 1.1133x over previous
import functools

import jax
import jax.numpy as jnp
from jax import lax
from jax.experimental import pallas as pl
from jax.experimental.pallas import tpu as pltpu
from jax.experimental.pallas import tpu_sc as plsc

N_NODES_C = 10000
IN_CH_C = 128
OUT_CH_C = 128
NNZ_C = 160000

_NC = 2
_NS = 16
_NW = _NC * _NS

_PER_W = NNZ_C // _NW
_ROWS128 = 40
_PAD_W = _ROWS128 * 128
_FLAT = N_NODES_C * IN_CH_C
_STRIPE = _FLAT // _NS
_ZB = 8000


def _sc_body(rows_hbm, cols_hbm, vals_hbm, out0_hbm, out1_hbm,
             shared, rows_v, cols_v, vals_v, flat2, vals2, zbuf,
             sem_in, sem_z, sem_sc):
    c = lax.axis_index("c")
    s = lax.axis_index("s")
    w = c * _NS + s
    base = w * _PER_W

    d_in = [
        pltpu.async_copy(rows_hbm.at[pl.ds(base, _PER_W)],
                         rows_v.at[pl.ds(0, _PER_W)], sem_in),
        pltpu.async_copy(cols_hbm.at[pl.ds(base, _PER_W)],
                         cols_v.at[pl.ds(0, _PER_W)], sem_in),
        pltpu.async_copy(vals_hbm.at[pl.ds(base, _PER_W)],
                         vals_v.at[pl.ds(0, _PER_W)], sem_in),
    ]

    zero16 = jnp.zeros((16,), jnp.float32)

    def _zb_body(i, _):
        for u in range(4):
            zbuf[pl.ds(pl.multiple_of(i * 64 + u * 16, 16), 16)] = zero16
        return _

    lax.fori_loop(0, _ZB // 64, _zb_body, None)
    d_z = [
        pltpu.async_copy(zbuf, shared.at[pl.ds(s * _STRIPE + k * _ZB, _ZB)],
                         sem_z)
        for k in range(_STRIPE // _ZB)
    ]

    for d in d_in:
        d.wait()
    iota16 = lax.iota(jnp.int32, 16)

    def _cvt_body(j, _):
        for k in range(8):
            off = pl.multiple_of(j * 128 + k * 16, 16)
            r = rows_v[pl.ds(off, 16)]
            cc = cols_v[pl.ds(off, 16)]
            flat2[j, pl.ds(k * 16, 16)] = r * IN_CH_C + cc
            vals2[j, pl.ds(k * 16, 16)] = vals_v[pl.ds(off, 16)]
        return _

    lax.fori_loop(0, _ROWS128 - 1, _cvt_body, None)
    jt = _ROWS128 - 1

    def _tail_body(k, _):
        off = pl.multiple_of(jt * 128 + k * 16, 16)
        ok = (k * 16 + iota16) < (_PER_W - jt * 128)
        r = rows_v[pl.ds(off, 16)]
        cc = cols_v[pl.ds(off, 16)]
        v = vals_v[pl.ds(off, 16)]
        flat2[jt, pl.ds(k * 16, 16)] = jnp.where(ok, r * IN_CH_C + cc, 0)
        vals2[jt, pl.ds(k * 16, 16)] = jnp.where(ok, v, jnp.float32(0.0))
        return _

    lax.fori_loop(0, 8, _tail_body, None)

    for d in d_z:
        d.wait()

    plsc.subcore_barrier()

    _PRE = 4
    d_pre = [
        pltpu.async_copy(vals2.at[j], shared.at[flat2.at[j]], sem_sc,
                         add=True)
        for j in range(_PRE)
    ]

    def _sc_scatter(j, _):
        d = pltpu.async_copy(vals2.at[j], shared.at[flat2.at[j]], sem_sc,
                             add=True)
        d.wait()
        return _

    lax.fori_loop(_PRE, _ROWS128, _sc_scatter, None)
    for d in d_pre:
        d.wait()

    plsc.subcore_barrier()

    @pl.when(c == 0)
    def _():
        pltpu.sync_copy(shared.at[pl.ds(s * _STRIPE, _STRIPE)],
                        out0_hbm.at[pl.ds(s * _STRIPE, _STRIPE)])

    @pl.when(c == 1)
    def _():
        pltpu.sync_copy(shared.at[pl.ds(s * _STRIPE, _STRIPE)],
                        out1_hbm.at[pl.ds(s * _STRIPE, _STRIPE)])


_densify = functools.partial(
    pl.kernel,
    out_type=[jax.ShapeDtypeStruct((_FLAT,), jnp.float32),
              jax.ShapeDtypeStruct((_FLAT,), jnp.float32)],
    mesh=plsc.VectorSubcoreMesh(core_axis_name="c", subcore_axis_name="s"),
    scratch_types=[
        pltpu.VMEM_SHARED((_FLAT,), jnp.float32),
        pltpu.VMEM((_PAD_W,), jnp.int32),
        pltpu.VMEM((_PAD_W,), jnp.int32),
        pltpu.VMEM((_PAD_W,), jnp.float32),
        pltpu.VMEM((_ROWS128, 128), jnp.int32),
        pltpu.VMEM((_ROWS128, 128), jnp.float32),
        pltpu.VMEM((_ZB,), jnp.float32),
        pltpu.SemaphoreType.DMA,
        pltpu.SemaphoreType.DMA,
        pltpu.SemaphoreType.DMA,
    ],
)(_sc_body)


def _mm_body(s0_ref, s1_ref, w_ref, o_ref):
    a = s0_ref[...] + s1_ref[...]
    o_ref[...] = jnp.dot(a, w_ref[...], preferred_element_type=jnp.float32)


_ROW_BLK = 5000


def _matmul(s0, s1, weight):
    return pl.pallas_call(
        _mm_body,
        grid=(N_NODES_C // _ROW_BLK,),
        in_specs=[
            pl.BlockSpec((_ROW_BLK, IN_CH_C), lambda i: (i, 0)),
            pl.BlockSpec((_ROW_BLK, IN_CH_C), lambda i: (i, 0)),
            pl.BlockSpec((IN_CH_C, OUT_CH_C), lambda i: (0, 0)),
        ],
        out_specs=pl.BlockSpec((_ROW_BLK, OUT_CH_C), lambda i: (i, 0)),
        out_shape=jax.ShapeDtypeStruct((N_NODES_C, OUT_CH_C), jnp.float32),
    )(s0, s1, weight)


def kernel(adj_indices, adj_values, feat_rows, feat_cols, feat_values, weight):
    del adj_indices, adj_values
    p0, p1 = _densify(feat_rows.astype(jnp.int32), feat_cols.astype(jnp.int32),
                      feat_values)
    s0 = p0.reshape(N_NODES_C, IN_CH_C)
    s1 = p1.reshape(N_NODES_C, IN_CH_C)
    return _matmul(s0, s1, weight)
